# baseline (speedup 1.0000x reference)
import jax
import jax.numpy as jnp
from jax.experimental import pallas as pl

_EPS = 1e-3
_H = 384
_W = 384
_C = 192
_HT = 16
_GRID = _H // _HT
_N = float(_H * _W)


def _conv_kernel(x_ref, halo_ref, w_ref, q_ref, s_ref):
    i = pl.program_id(0)
    xw = jnp.concatenate([halo_ref[0, :, 0:1], x_ref[...], halo_ref[0, :, 1:2]],
                         axis=1)
    xb = xw.astype(jnp.bfloat16)
    xt = jnp.transpose(xb, (1, 2, 0))
    z = jnp.zeros((_HT + 2, 1, _C), jnp.bfloat16)
    xfull = jnp.concatenate([z, xt, z], axis=1)
    acc = jnp.zeros((_HT, _W, _C), jnp.float32)
    for dh in range(3):
        for dw in range(3):
            xs = xfull[dh:dh + _HT, dw:dw + _W, :]
            wt = w_ref[dh * 3 + dw]
            acc = acc + jax.lax.dot_general(
                xs, wt, (((2,), (0,)), ((), ())),
                preferred_element_type=jnp.float32)
    q_ref[...] = acc
    st = jnp.stack([jnp.sum(acc, axis=(0, 1)),
                    jnp.sum(acc * acc, axis=(0, 1))], axis=0)

    @pl.when(i == 0)
    def _():
        s_ref[...] = jnp.zeros_like(s_ref)

    s_ref[...] += st


def _bn_kernel(q_ref, s_ref, g_ref, be_ref, y_ref):
    s = s_ref[0:1, :]
    s2 = s_ref[1:2, :]
    mean = s * (1.0 / _N)
    var = s2 * (1.0 / _N) - mean * mean
    inv = jax.lax.rsqrt(var + _EPS)
    scale = g_ref[...] * inv
    shift = be_ref[...] - mean * scale
    y = jnp.maximum(q_ref[...] * scale[None] + shift[None], 0.0)
    y_ref[...] = jnp.transpose(y, (2, 0, 1))


def kernel(inp_NHWC, active_block_indices, bin_counts, W, b, gamma, beta):
    del active_block_indices, bin_counts, b
    x = inp_NHWC[0]
    top = jnp.concatenate([jnp.zeros((_C, 1, _W), x.dtype),
                           x[:, _HT - 1:_H - _HT:_HT, :]], axis=1)
    bot = jnp.concatenate([x[:, _HT:_H - _HT + 1:_HT, :],
                           jnp.zeros((_C, 1, _W), x.dtype)], axis=1)
    halo = jnp.transpose(jnp.stack([top, bot], axis=2), (1, 0, 2, 3))
    w9 = jnp.transpose(W, (2, 3, 1, 0)).reshape(9, _C, _C).astype(jnp.bfloat16)

    q, stats = pl.pallas_call(
        _conv_kernel,
        grid=(_GRID,),
        in_specs=[
            pl.BlockSpec((_C, _HT, _W), lambda i: (0, i, 0)),
            pl.BlockSpec((1, _C, 2, _W), lambda i: (i, 0, 0, 0)),
            pl.BlockSpec((9, _C, _C), lambda i: (0, 0, 0)),
        ],
        out_specs=[
            pl.BlockSpec((_HT, _W, _C), lambda i: (i, 0, 0)),
            pl.BlockSpec((2, _C), lambda i: (0, 0)),
        ],
        out_shape=[
            jax.ShapeDtypeStruct((_H, _W, _C), jnp.float32),
            jax.ShapeDtypeStruct((2, _C), jnp.float32),
        ],
    )(x, halo, w9)

    y = pl.pallas_call(
        _bn_kernel,
        grid=(_GRID,),
        in_specs=[
            pl.BlockSpec((_HT, _W, _C), lambda i: (i, 0, 0)),
            pl.BlockSpec((2, _C), lambda i: (0, 0)),
            pl.BlockSpec((1, _C), lambda i: (0, 0)),
            pl.BlockSpec((1, _C), lambda i: (0, 0)),
        ],
        out_specs=pl.BlockSpec((_C, _HT, _W), lambda i: (0, i, 0)),
        out_shape=jax.ShapeDtypeStruct((_C, _H, _W), jnp.float32),
    )(q, stats, gamma.reshape(1, _C), beta.reshape(1, _C))

    return y[None]


# speedup vs baseline: 1.2063x; 1.2063x over previous
import jax
import jax.numpy as jnp
from jax.experimental import pallas as pl

_EPS = 1e-3
_H = 384
_W = 384
_C = 192
_HT = 16
_GRID = _H // _HT
_M = _HT * _W
_N = float(_H * _W)


def _conv_kernel(x_ref, halo_ref, w_ref, q_ref, s_ref):
    i = pl.program_id(0)
    xcat = jnp.concatenate([halo_ref[0, 0], x_ref[...], halo_ref[0, 1]],
                           axis=1).astype(jnp.bfloat16)
    xt = jnp.transpose(xcat, (1, 0))
    xfull = xt.reshape(_HT + 2, _W, _C)
    z = jnp.zeros((_HT + 2, 1, _C), jnp.bfloat16)
    xw = jnp.concatenate([z, xfull, z], axis=1)
    acc = jnp.zeros((_HT, _W, _C), jnp.float32)
    for dh in range(3):
        for dw in range(3):
            xs = xw[dh:dh + _HT, dw:dw + _W, :]
            wt = w_ref[dh * 3 + dw]
            acc = acc + jax.lax.dot_general(
                xs, wt, (((2,), (0,)), ((), ())),
                preferred_element_type=jnp.float32)
    acc2 = acc.reshape(_M, _C)
    q_ref[...] = acc2
    st = jnp.stack([jnp.sum(acc2, axis=0),
                    jnp.sum(acc2 * acc2, axis=0)], axis=0)

    @pl.when(i == 0)
    def _():
        s_ref[...] = jnp.zeros_like(s_ref)

    s_ref[...] += st


def _bn_kernel(q_ref, s_ref, g_ref, be_ref, y_ref):
    s = s_ref[0:1, :]
    s2 = s_ref[1:2, :]
    mean = s * (1.0 / _N)
    var = s2 * (1.0 / _N) - mean * mean
    inv = jax.lax.rsqrt(var + _EPS)
    scale = g_ref[...] * inv
    shift = be_ref[...] - mean * scale
    y = jnp.maximum(q_ref[...] * scale + shift, 0.0)
    y_ref[...] = jnp.transpose(y, (1, 0))


def kernel(inp_NHWC, active_block_indices, bin_counts, W, b, gamma, beta):
    del active_block_indices, bin_counts, b
    x = inp_NHWC[0]
    x2 = x.reshape(_C, _H * _W)
    top = jnp.concatenate([jnp.zeros((_C, 1, _W), x.dtype),
                           x[:, _HT - 1:_H - _HT:_HT, :]], axis=1)
    bot = jnp.concatenate([x[:, _HT:_H - _HT + 1:_HT, :],
                           jnp.zeros((_C, 1, _W), x.dtype)], axis=1)
    halo = jnp.transpose(jnp.stack([top, bot], axis=0), (2, 0, 1, 3))
    w9 = jnp.transpose(W, (2, 3, 1, 0)).reshape(9, _C, _C).astype(jnp.bfloat16)

    q, stats = pl.pallas_call(
        _conv_kernel,
        grid=(_GRID,),
        in_specs=[
            pl.BlockSpec((_C, _M), lambda i: (0, i)),
            pl.BlockSpec((1, 2, _C, _W), lambda i: (i, 0, 0, 0)),
            pl.BlockSpec((9, _C, _C), lambda i: (0, 0, 0)),
        ],
        out_specs=[
            pl.BlockSpec((_M, _C), lambda i: (i, 0)),
            pl.BlockSpec((2, _C), lambda i: (0, 0)),
        ],
        out_shape=[
            jax.ShapeDtypeStruct((_H * _W, _C), jnp.float32),
            jax.ShapeDtypeStruct((2, _C), jnp.float32),
        ],
    )(x2, halo, w9)

    y = pl.pallas_call(
        _bn_kernel,
        grid=(_GRID,),
        in_specs=[
            pl.BlockSpec((_M, _C), lambda i: (i, 0)),
            pl.BlockSpec((2, _C), lambda i: (0, 0)),
            pl.BlockSpec((1, _C), lambda i: (0, 0)),
            pl.BlockSpec((1, _C), lambda i: (0, 0)),
        ],
        out_specs=pl.BlockSpec((_C, _M), lambda i: (0, i)),
        out_shape=jax.ShapeDtypeStruct((_C, _H * _W), jnp.float32),
    )(q, stats, gamma.reshape(1, _C), beta.reshape(1, _C))

    return y.reshape(_C, _H, _W)[None]


# speedup vs baseline: 1.4872x; 1.2329x over previous
import jax
import jax.numpy as jnp
from jax.experimental import pallas as pl

_EPS = 1e-3
_H = 384
_W = 384
_C = 192
_HT = 16
_GRID = _H // _HT
_M = _HT * _W
_N = float(_H * _W)


def _conv_kernel(x_ref, top_ref, bot_ref, w_ref, q_ref, s_ref):
    i = pl.program_id(0)
    top = jnp.where(i > 0, top_ref[...], 0.0)
    bot = jnp.where(i < _GRID - 1, bot_ref[...], 0.0)
    xcat = jnp.concatenate([top, x_ref[...], bot],
                           axis=1).astype(jnp.bfloat16)
    xt = jnp.transpose(xcat, (1, 0))
    xfull = xt.reshape(_HT + 2, _W, _C)
    z = jnp.zeros((_HT + 2, 1, _C), jnp.bfloat16)
    xw = jnp.concatenate([z, xfull, z], axis=1)
    acc = jnp.zeros((_HT, _W, _C), jnp.float32)
    for dh in range(3):
        for dw in range(3):
            xs = xw[dh:dh + _HT, dw:dw + _W, :]
            wt = w_ref[dh * 3 + dw]
            acc = acc + jax.lax.dot_general(
                xs, wt, (((2,), (0,)), ((), ())),
                preferred_element_type=jnp.float32)
    acc2 = acc.reshape(_M, _C)
    q_ref[...] = acc2
    st = jnp.stack([jnp.sum(acc2, axis=0),
                    jnp.sum(acc2 * acc2, axis=0)], axis=0)

    @pl.when(i == 0)
    def _():
        s_ref[...] = jnp.zeros_like(s_ref)

    s_ref[...] += st


def _bn_kernel(q_ref, s_ref, g_ref, be_ref, y_ref):
    s = s_ref[0:1, :]
    s2 = s_ref[1:2, :]
    mean = s * (1.0 / _N)
    var = s2 * (1.0 / _N) - mean * mean
    inv = jax.lax.rsqrt(var + _EPS)
    scale = g_ref[...] * inv
    shift = be_ref[...] - mean * scale
    y = jnp.maximum(q_ref[...] * scale + shift, 0.0)
    y_ref[...] = jnp.transpose(y, (1, 0))


def kernel(inp_NHWC, active_block_indices, bin_counts, W, b, gamma, beta):
    del active_block_indices, bin_counts, b
    x2 = inp_NHWC[0].reshape(_C, _H * _W)
    w9 = jnp.transpose(W, (2, 3, 1, 0)).reshape(9, _C, _C).astype(jnp.bfloat16)

    q, stats = pl.pallas_call(
        _conv_kernel,
        grid=(_GRID,),
        in_specs=[
            pl.BlockSpec((_C, _M), lambda i: (0, i)),
            pl.BlockSpec((_C, _W), lambda i: (0, jnp.maximum(16 * i - 1, 0))),
            pl.BlockSpec((_C, _W), lambda i: (0, jnp.minimum(16 * i + 16, _H - 1))),
            pl.BlockSpec((9, _C, _C), lambda i: (0, 0, 0)),
        ],
        out_specs=[
            pl.BlockSpec((_M, _C), lambda i: (i, 0)),
            pl.BlockSpec((2, _C), lambda i: (0, 0)),
        ],
        out_shape=[
            jax.ShapeDtypeStruct((_H * _W, _C), jnp.float32),
            jax.ShapeDtypeStruct((2, _C), jnp.float32),
        ],
    )(x2, x2, x2, w9)

    y = pl.pallas_call(
        _bn_kernel,
        grid=(_GRID,),
        in_specs=[
            pl.BlockSpec((_M, _C), lambda i: (i, 0)),
            pl.BlockSpec((2, _C), lambda i: (0, 0)),
            pl.BlockSpec((1, _C), lambda i: (0, 0)),
            pl.BlockSpec((1, _C), lambda i: (0, 0)),
        ],
        out_specs=pl.BlockSpec((_C, _M), lambda i: (0, i)),
        out_shape=jax.ShapeDtypeStruct((_C, _H * _W), jnp.float32),
    )(q, stats, gamma.reshape(1, _C), beta.reshape(1, _C))

    return y.reshape(_C, _H, _W)[None]


# speedup vs baseline: 2.1483x; 1.4445x over previous
import jax
import jax.numpy as jnp
from jax.experimental import pallas as pl

_EPS = 1e-3
_H = 384
_W = 384
_C = 192
_HT = 8
_GRID = _H // _HT
_M = _HT * _W
_N = float(_H * _W)


def _conv_kernel(x_ref, top_ref, bot_ref, w_ref, q_ref, s_ref):
    i = pl.program_id(0)
    top8 = top_ref[...].reshape(_C, 8 * _W)
    bot8 = bot_ref[...].reshape(_C, 8 * _W)
    top = jnp.where(i > 0, top8[:, 7 * _W:8 * _W], 0.0)
    bot = jnp.where(i < _GRID - 1, bot8[:, 0:_W], 0.0)
    xm = x_ref[...].reshape(_C, _M)
    xcat = jnp.concatenate([top, xm, bot], axis=1).astype(jnp.bfloat16)
    xt = jnp.transpose(xcat, (1, 0))
    xfull = xt.reshape(_HT + 2, _W, _C)
    z = jnp.zeros((_HT + 2, 1, _C), jnp.bfloat16)
    xw = jnp.concatenate([z, xfull, z], axis=1)
    acc = jnp.zeros((_HT, _W, _C), jnp.float32)
    for dh in range(3):
        for dw in range(3):
            xs = xw[dh:dh + _HT, dw:dw + _W, :]
            wt = w_ref[dh * 3 + dw]
            acc = acc + jax.lax.dot_general(
                xs, wt, (((2,), (0,)), ((), ())),
                preferred_element_type=jnp.float32)
    acc2 = acc.reshape(_M, _C)
    q_ref[...] = acc2
    st = jnp.stack([jnp.sum(acc2, axis=0),
                    jnp.sum(acc2 * acc2, axis=0)], axis=0)

    @pl.when(i == 0)
    def _():
        s_ref[...] = jnp.zeros_like(s_ref)

    s_ref[...] += st


def _bn_kernel(q_ref, s_ref, g_ref, be_ref, y_ref):
    s = s_ref[0:1, :]
    s2 = s_ref[1:2, :]
    mean = s * (1.0 / _N)
    var = s2 * (1.0 / _N) - mean * mean
    inv = jax.lax.rsqrt(var + _EPS)
    scale = g_ref[...] * inv
    shift = be_ref[...] - mean * scale
    y = jnp.maximum(q_ref[...] * scale + shift, 0.0)
    y_ref[...] = jnp.transpose(y, (1, 0)).reshape(_C, _HT, _W)


def kernel(inp_NHWC, active_block_indices, bin_counts, W, b, gamma, beta):
    del active_block_indices, bin_counts, b
    x = inp_NHWC[0]
    w9 = jnp.transpose(W, (2, 3, 1, 0)).reshape(9, _C, _C).astype(jnp.bfloat16)

    q, stats = pl.pallas_call(
        _conv_kernel,
        grid=(_GRID,),
        in_specs=[
            pl.BlockSpec((_C, _HT, _W), lambda i: (0, i, 0)),
            pl.BlockSpec((_C, 8, _W), lambda i: (0, jnp.maximum(i - 1, 0), 0)),
            pl.BlockSpec((_C, 8, _W), lambda i: (0, jnp.minimum(i + 1, _H // 8 - 1), 0)),
            pl.BlockSpec((9, _C, _C), lambda i: (0, 0, 0)),
        ],
        out_specs=[
            pl.BlockSpec((_M, _C), lambda i: (i, 0)),
            pl.BlockSpec((2, _C), lambda i: (0, 0)),
        ],
        out_shape=[
            jax.ShapeDtypeStruct((_H * _W, _C), jnp.float32),
            jax.ShapeDtypeStruct((2, _C), jnp.float32),
        ],
    )(x, x, x, w9)

    y = pl.pallas_call(
        _bn_kernel,
        grid=(_GRID,),
        in_specs=[
            pl.BlockSpec((_M, _C), lambda i: (i, 0)),
            pl.BlockSpec((2, _C), lambda i: (0, 0)),
            pl.BlockSpec((1, _C), lambda i: (0, 0)),
            pl.BlockSpec((1, _C), lambda i: (0, 0)),
        ],
        out_specs=pl.BlockSpec((_C, _HT, _W), lambda i: (0, i, 0)),
        out_shape=jax.ShapeDtypeStruct((_C, _H, _W), jnp.float32),
    )(q, stats, gamma.reshape(1, _C), beta.reshape(1, _C))

    return y[None]


# speedup vs baseline: 2.2513x; 1.0479x over previous
import jax
import jax.numpy as jnp
from jax.experimental import pallas as pl

_EPS = 1e-3
_H = 384
_W = 384
_C = 192
_HT = 8
_GRID = _H // _HT
_M = _HT * _W
_N = float(_H * _W)


def _conv_kernel(x_ref, top_ref, bot_ref, w_ref, q_ref, s_ref):
    i = pl.program_id(0)
    top8 = top_ref[...].reshape(_C, 8 * _W)
    bot8 = bot_ref[...].reshape(_C, 8 * _W)
    top = jnp.where(i > 0, top8[:, 7 * _W:8 * _W], 0.0).astype(jnp.bfloat16)
    bot = jnp.where(i < _GRID - 1, bot8[:, 0:_W], 0.0).astype(jnp.bfloat16)
    xm = x_ref[...].reshape(_C, _M).astype(jnp.bfloat16)
    xcat = jnp.concatenate([top, xm, bot], axis=1)
    xt = jnp.transpose(xcat, (1, 0))
    xfull = xt.reshape(_HT + 2, _W, _C)
    z = jnp.zeros((_HT + 2, 1, _C), jnp.bfloat16)
    xw = jnp.concatenate([z, xfull, z], axis=1)
    acc = jnp.zeros((_HT, _W, _C), jnp.float32)
    for dh in range(3):
        for dw in range(3):
            xs = xw[dh:dh + _HT, dw:dw + _W, :]
            wt = w_ref[dh * 3 + dw]
            acc = acc + jax.lax.dot_general(
                xs, wt, (((2,), (0,)), ((), ())),
                preferred_element_type=jnp.float32)
    acc2 = acc.reshape(_M, _C)
    q_ref[...] = acc2.astype(jnp.bfloat16)
    st = jnp.stack([jnp.sum(acc2, axis=0),
                    jnp.sum(acc2 * acc2, axis=0)], axis=0)

    @pl.when(i == 0)
    def _():
        s_ref[...] = jnp.zeros_like(s_ref)

    s_ref[...] += st


def _bn_kernel(q_ref, s_ref, g_ref, be_ref, y_ref):
    s = s_ref[0:1, :]
    s2 = s_ref[1:2, :]
    mean = s * (1.0 / _N)
    var = s2 * (1.0 / _N) - mean * mean
    inv = jax.lax.rsqrt(var + _EPS)
    scale = g_ref[...] * inv
    shift = be_ref[...] - mean * scale
    q = q_ref[...].astype(jnp.float32)
    y = jnp.maximum(q * scale + shift, 0.0)
    y_ref[...] = jnp.transpose(y, (1, 0)).reshape(_C, _HT, _W)


def kernel(inp_NHWC, active_block_indices, bin_counts, W, b, gamma, beta):
    del active_block_indices, bin_counts, b
    x = inp_NHWC[0]
    w9 = jnp.transpose(W, (2, 3, 1, 0)).reshape(9, _C, _C).astype(jnp.bfloat16)

    q, stats = pl.pallas_call(
        _conv_kernel,
        grid=(_GRID,),
        in_specs=[
            pl.BlockSpec((_C, _HT, _W), lambda i: (0, i, 0)),
            pl.BlockSpec((_C, 8, _W), lambda i: (0, jnp.maximum(i - 1, 0), 0)),
            pl.BlockSpec((_C, 8, _W), lambda i: (0, jnp.minimum(i + 1, _H // 8 - 1), 0)),
            pl.BlockSpec((9, _C, _C), lambda i: (0, 0, 0)),
        ],
        out_specs=[
            pl.BlockSpec((_M, _C), lambda i: (i, 0)),
            pl.BlockSpec((2, _C), lambda i: (0, 0)),
        ],
        out_shape=[
            jax.ShapeDtypeStruct((_H * _W, _C), jnp.bfloat16),
            jax.ShapeDtypeStruct((2, _C), jnp.float32),
        ],
    )(x, x, x, w9)

    y = pl.pallas_call(
        _bn_kernel,
        grid=(_GRID,),
        in_specs=[
            pl.BlockSpec((_M, _C), lambda i: (i, 0)),
            pl.BlockSpec((2, _C), lambda i: (0, 0)),
            pl.BlockSpec((1, _C), lambda i: (0, 0)),
            pl.BlockSpec((1, _C), lambda i: (0, 0)),
        ],
        out_specs=pl.BlockSpec((_C, _HT, _W), lambda i: (0, i, 0)),
        out_shape=jax.ShapeDtypeStruct((_C, _H, _W), jnp.float32),
    )(q, stats, gamma.reshape(1, _C), beta.reshape(1, _C))

    return y[None]


# speedup vs baseline: 2.3576x; 1.0472x over previous
import jax
import jax.numpy as jnp
from jax.experimental import pallas as pl

_EPS = 1e-3
_H = 384
_W = 384
_C = 192
_HT = 16
_GRID = _H // _HT
_M = _HT * _W
_N = float(_H * _W)


def _conv_kernel(x_ref, top_ref, bot_ref, w_ref, q_ref, s_ref):
    i = pl.program_id(0)
    top8 = top_ref[...].reshape(_C, 8 * _W)
    bot8 = bot_ref[...].reshape(_C, 8 * _W)
    top = jnp.where(i > 0, top8[:, 7 * _W:8 * _W], 0.0).astype(jnp.bfloat16)
    bot = jnp.where(i < _GRID - 1, bot8[:, 0:_W], 0.0).astype(jnp.bfloat16)
    xm = x_ref[...].reshape(_C, _M).astype(jnp.bfloat16)
    xcat = jnp.concatenate([top, xm, bot], axis=1)
    xt = jnp.transpose(xcat, (1, 0))
    xfull = xt.reshape(_HT + 2, _W, _C)
    z = jnp.zeros((_HT + 2, 1, _C), jnp.bfloat16)
    xw = jnp.concatenate([z, xfull, z], axis=1)
    acc = jnp.zeros((_HT, _W, _C), jnp.float32)
    for dh in range(3):
        for dw in range(3):
            xs = xw[dh:dh + _HT, dw:dw + _W, :]
            wt = w_ref[dh * 3 + dw]
            acc = acc + jax.lax.dot_general(
                xs, wt, (((2,), (0,)), ((), ())),
                preferred_element_type=jnp.float32)
    acc2 = acc.reshape(_M, _C)
    q_ref[...] = acc2.astype(jnp.bfloat16)
    st = jnp.stack([jnp.sum(acc2, axis=0),
                    jnp.sum(acc2 * acc2, axis=0)], axis=0)

    @pl.when(i == 0)
    def _():
        s_ref[...] = jnp.zeros_like(s_ref)

    s_ref[...] += st


def _bn_kernel(q_ref, s_ref, g_ref, be_ref, y_ref):
    s = s_ref[0:1, :]
    s2 = s_ref[1:2, :]
    mean = s * (1.0 / _N)
    var = s2 * (1.0 / _N) - mean * mean
    inv = jax.lax.rsqrt(var + _EPS)
    scale = g_ref[...] * inv
    shift = be_ref[...] - mean * scale
    q = q_ref[...].astype(jnp.float32)
    y = jnp.maximum(q * scale + shift, 0.0)
    y_ref[...] = jnp.transpose(y, (1, 0)).reshape(_C, _HT, _W)


def kernel(inp_NHWC, active_block_indices, bin_counts, W, b, gamma, beta):
    del active_block_indices, bin_counts, b
    x = inp_NHWC[0]
    w9 = jnp.transpose(W, (2, 3, 1, 0)).reshape(9, _C, _C).astype(jnp.bfloat16)

    q, stats = pl.pallas_call(
        _conv_kernel,
        grid=(_GRID,),
        in_specs=[
            pl.BlockSpec((_C, _HT, _W), lambda i: (0, i, 0)),
            pl.BlockSpec((_C, 8, _W), lambda i: (0, jnp.maximum(2 * i - 1, 0), 0)),
            pl.BlockSpec((_C, 8, _W), lambda i: (0, jnp.minimum(2 * i + 2, _H // 8 - 1), 0)),
            pl.BlockSpec((9, _C, _C), lambda i: (0, 0, 0)),
        ],
        out_specs=[
            pl.BlockSpec((_M, _C), lambda i: (i, 0)),
            pl.BlockSpec((2, _C), lambda i: (0, 0)),
        ],
        out_shape=[
            jax.ShapeDtypeStruct((_H * _W, _C), jnp.bfloat16),
            jax.ShapeDtypeStruct((2, _C), jnp.float32),
        ],
    )(x, x, x, w9)

    y = pl.pallas_call(
        _bn_kernel,
        grid=(_GRID,),
        in_specs=[
            pl.BlockSpec((_M, _C), lambda i: (i, 0)),
            pl.BlockSpec((2, _C), lambda i: (0, 0)),
            pl.BlockSpec((1, _C), lambda i: (0, 0)),
            pl.BlockSpec((1, _C), lambda i: (0, 0)),
        ],
        out_specs=pl.BlockSpec((_C, _HT, _W), lambda i: (0, i, 0)),
        out_shape=jax.ShapeDtypeStruct((_C, _H, _W), jnp.float32),
    )(q, stats, gamma.reshape(1, _C), beta.reshape(1, _C))

    return y[None]
